# baseline (speedup 1.0000x reference)
import jax
import jax.numpy as jnp
import numpy as np
from jax.experimental import pallas as pl
from jax.experimental.pallas import tpu as pltpu

_IDX = np.array([0, 2, 3, 5, 7, 8, 10, 12, 13, 15, 17, 20, 22, 25, 27, 30,
                 33, 36, 40, 44, 48, 52, 57, 62], dtype=np.int32)

_SEL = np.zeros((64, 24), dtype=np.float32)
_SEL[_IDX, np.arange(24)] = 1.0


def _tc_body(x_ref, sel_ref, o_ref):
    o_ref[...] = jax.lax.dot(x_ref[...], sel_ref[...],
                             preferred_element_type=jnp.float32)


def kernel(inputs):
    B, C, R, D = inputs.shape
    rows = B * C * R
    x2 = inputs.reshape(rows, D)
    block = 8192
    out2 = pl.pallas_call(
        _tc_body,
        grid=(rows // block,),
        in_specs=[pl.BlockSpec((block, D), lambda i: (i, 0)),
                  pl.BlockSpec((D, 24), lambda i: (0, 0))],
        out_specs=pl.BlockSpec((block, 24), lambda i: (i, 0)),
        out_shape=jax.ShapeDtypeStruct((rows, 24), jnp.float32),
    )(x2, jnp.asarray(_SEL))
    return out2.reshape(B, C, R, 24)
